# baseline (device time: 40901 ns/iter reference)
import jax
import jax.numpy as jnp
from jax import lax
from jax.experimental import pallas as pl
from jax.experimental.pallas import tpu as pltpu

N_DEV = 8
B, SQ, D, SKV, DH = 2, 256, 768, 512, 64
HQ_LOC = 8
ROWS = B * SQ
CH = ROWS // N_DEV

F32 = jnp.float32
BF16 = jnp.bfloat16


def kernel(x, Wq, Wo, K_ext, V_ext):
    def body(x_ref, wq_ref, wo_ref, k_ref, v_ref, out_ref,
             x_v, wq_v, wo_v, kv_v,
             part_ref, red_ref, rs_buf, ag_buf,
             ld_sems, rs_send, rs_recv, ag_send, ag_recv):
        me = lax.axis_index("i")

        ld_x = pltpu.make_async_copy(x_ref, x_v, ld_sems.at[0])
        ld_wq = pltpu.make_async_copy(wq_ref, wq_v, ld_sems.at[1])
        ld_k = pltpu.make_async_copy(
            k_ref.at[:, :, pl.ds(2 * me, 2), :], kv_v.at[0], ld_sems.at[2])
        ld_v = pltpu.make_async_copy(
            v_ref.at[:, :, pl.ds(2 * me, 2), :], kv_v.at[1], ld_sems.at[3])
        ld_wo = pltpu.make_async_copy(wo_ref, wo_v, ld_sems.at[4])
        for ld in (ld_x, ld_wq, ld_k, ld_v, ld_wo):
            ld.start()

        bar = pltpu.get_barrier_semaphore()
        for d in range(1, N_DEV):
            t = lax.rem(me + d, N_DEV)
            pl.semaphore_signal(bar, inc=1, device_id=(t,),
                                device_id_type=pl.DeviceIdType.MESH)
        pl.semaphore_wait(bar, N_DEV - 1)

        ld_x.wait()
        ld_wq.wait()
        xf = x_v[...].reshape(ROWS, D).astype(BF16)
        q = jnp.dot(xf, wq_v[...].astype(BF16), preferred_element_type=F32)

        ld_wo.wait()
        sends = []
        rs_rdmas = []
        for d in range(1, N_DEV):
            t = lax.rem(me + d, N_DEV)
            rdma = pltpu.make_async_remote_copy(
                src_ref=part_ref.at[pl.ds(CH * t, CH), :],
                dst_ref=rs_buf.at[N_DEV - d],
                send_sem=rs_send.at[d - 1],
                recv_sem=rs_recv.at[N_DEV - d],
                device_id=(t,),
                device_id_type=pl.DeviceIdType.MESH,
            )
            rs_rdmas.append((t, rdma))
            sends.append(rdma)

        ld_k.wait()
        ld_v.wait()
        for b in range(B):
            qb = q[b * SQ:(b + 1) * SQ, :].astype(BF16)
            kb = kv_v[0, b].astype(BF16)
            vb = kv_v[1, b].astype(BF16)
            head_outs = []
            for h in range(HQ_LOC):
                qh = qb[:, h * DH:(h + 1) * DH]
                kg = kb[:, h // 4, :]
                vg = vb[:, h // 4, :]
                s = lax.dot_general(
                    qh, kg, (((1,), (1,)), ((), ())),
                    preferred_element_type=F32) * 0.125
                p = jnp.exp(s)
                l = jnp.sum(p, axis=-1, keepdims=True)
                o = jnp.dot(p.astype(BF16), vg,
                            preferred_element_type=F32) / l
                head_outs.append(o.astype(BF16))
            ob = jnp.concatenate(head_outs, axis=1)
            part_ref[pl.ds(b * SQ, SQ), :] = jnp.dot(
                ob, wo_v[...].astype(BF16),
                preferred_element_type=F32).astype(BF16)
            for t, rdma in rs_rdmas:
                lo, hi = b * SQ // CH, (b + 1) * SQ // CH
                pl.when(jnp.logical_and(t >= lo, t < hi))(rdma.start)

        red = part_ref[pl.ds(CH * me, CH), :].astype(F32)
        for k in range(1, N_DEV):
            recv = pltpu.make_async_remote_copy(
                src_ref=rs_buf.at[k], dst_ref=rs_buf.at[k],
                send_sem=rs_send.at[0], recv_sem=rs_recv.at[k],
                device_id=(me,), device_id_type=pl.DeviceIdType.MESH,
            )
            recv.wait_recv()
            red = red + rs_buf[k].astype(F32)
        red_ref[...] = red.astype(BF16)

        for d in range(1, N_DEV):
            t = lax.rem(me + d, N_DEV)
            rdma = pltpu.make_async_remote_copy(
                src_ref=red_ref,
                dst_ref=ag_buf.at[N_DEV - d],
                send_sem=ag_send.at[d - 1],
                recv_sem=ag_recv.at[N_DEV - d],
                device_id=(t,),
                device_id_type=pl.DeviceIdType.MESH,
            )
            rdma.start()
            sends.append(rdma)

        out_ref[pl.ds(CH * me, CH), :] = red

        for k in range(1, N_DEV):
            recv = pltpu.make_async_remote_copy(
                src_ref=red_ref, dst_ref=ag_buf.at[k],
                send_sem=ag_send.at[0], recv_sem=ag_recv.at[k],
                device_id=(me,), device_id_type=pl.DeviceIdType.MESH,
            )
            recv.wait_recv()
            p = lax.rem(me + k, N_DEV)
            out_ref[pl.ds(CH * p, CH), :] = ag_buf[k].astype(F32)

        for rdma in sends:
            rdma.wait_send()

    out = pl.pallas_call(
        body,
        out_shape=jax.ShapeDtypeStruct((ROWS, D), F32),
        in_specs=[pl.BlockSpec(memory_space=pl.ANY)] * 5,
        out_specs=pl.BlockSpec(memory_space=pltpu.VMEM),
        scratch_shapes=[
            pltpu.VMEM((B, SQ, D), F32),
            pltpu.VMEM((D, ROWS), F32),
            pltpu.VMEM((ROWS, D), F32),
            pltpu.VMEM((2, B, SKV, 2, DH), F32),
            pltpu.VMEM((ROWS, D), BF16),
            pltpu.VMEM((CH, D), BF16),
            pltpu.VMEM((N_DEV, CH, D), BF16),
            pltpu.VMEM((N_DEV, CH, D), BF16),
            pltpu.SemaphoreType.DMA((5,)),
            pltpu.SemaphoreType.DMA((N_DEV - 1,)),
            pltpu.SemaphoreType.DMA((N_DEV,)),
            pltpu.SemaphoreType.DMA((N_DEV - 1,)),
            pltpu.SemaphoreType.DMA((N_DEV,)),
        ],
        compiler_params=pltpu.CompilerParams(collective_id=0),
    )(x, Wq, Wo, K_ext, V_ext)
    return out.reshape(B, SQ, D)


# device time: 36162 ns/iter; 1.1310x vs baseline; 1.1310x over previous
import jax
import jax.numpy as jnp
from jax import lax
from jax.experimental import pallas as pl
from jax.experimental.pallas import tpu as pltpu

N_DEV = 8
B, SQ, D, SKV, DH = 2, 256, 768, 512, 64
HQ_LOC = 8
ROWS = B * SQ
CH = ROWS // N_DEV

F32 = jnp.float32
BF16 = jnp.bfloat16


def kernel(x, Wq, Wo, K_ext, V_ext):
    def body(x_ref, wq_ref, wo_ref, k_ref, v_ref, out_ref,
             x_v, wq_v, wo_v, kv_v,
             part_ref, red_ref, rs_buf, ag_buf,
             ld_sems, rs_send, rs_recv, ag_send, ag_recv):
        me = lax.axis_index("i")

        ld_x = pltpu.make_async_copy(x_ref, x_v, ld_sems.at[0])
        ld_wq = pltpu.make_async_copy(wq_ref, wq_v, ld_sems.at[1])
        ld_k = pltpu.make_async_copy(
            k_ref.at[:, :, pl.ds(2 * me, 2), :], kv_v.at[0], ld_sems.at[2])
        ld_v = pltpu.make_async_copy(
            v_ref.at[:, :, pl.ds(2 * me, 2), :], kv_v.at[1], ld_sems.at[3])
        ld_wo = pltpu.make_async_copy(wo_ref, wo_v, ld_sems.at[4])
        for ld in (ld_x, ld_wq, ld_k, ld_v, ld_wo):
            ld.start()

        bar = pltpu.get_barrier_semaphore()
        for d in range(1, N_DEV):
            t = lax.rem(me + d, N_DEV)
            pl.semaphore_signal(bar, inc=1, device_id=(t,),
                                device_id_type=pl.DeviceIdType.MESH)
        pl.semaphore_wait(bar, N_DEV - 1)

        ld_x.wait()
        ld_wq.wait()
        xf = x_v[...].reshape(ROWS, D).astype(BF16)
        q = jnp.dot(xf, wq_v[...].astype(BF16), preferred_element_type=F32)

        ld_wo.wait()
        sends = []
        rs_rdmas = []
        for d in range(1, N_DEV):
            t = lax.rem(me + d, N_DEV)
            rdma = pltpu.make_async_remote_copy(
                src_ref=part_ref.at[pl.ds(CH * t, CH), :],
                dst_ref=rs_buf.at[N_DEV - d],
                send_sem=rs_send.at[d - 1],
                recv_sem=rs_recv.at[N_DEV - d],
                device_id=(t,),
                device_id_type=pl.DeviceIdType.MESH,
            )
            rs_rdmas.append((t, rdma))
            sends.append(rdma)

        ld_k.wait()
        ld_v.wait()
        for b in range(B):
            qb = q[b * SQ:(b + 1) * SQ, :].astype(BF16)
            kb = kv_v[0, b].astype(BF16)
            vb = kv_v[1, b].astype(BF16)
            grp_outs = []
            for g in range(2):
                q4 = jnp.concatenate(
                    [qb[:, (4 * g + i) * DH:(4 * g + i + 1) * DH]
                     for i in range(4)], axis=0)
                kg = kb[:, g, :]
                vg = vb[:, g, :]
                s = lax.dot_general(
                    q4, kg, (((1,), (1,)), ((), ())),
                    preferred_element_type=F32) * 0.125
                p = jnp.exp(s)
                l = jnp.sum(p, axis=-1, keepdims=True)
                o4 = jnp.dot(p.astype(BF16), vg,
                             preferred_element_type=F32) / l
                o4 = o4.astype(BF16)
                grp_outs.append(jnp.concatenate(
                    [o4[i * SQ:(i + 1) * SQ, :] for i in range(4)], axis=1))
            ob = jnp.concatenate(grp_outs, axis=1)
            part_ref[pl.ds(b * SQ, SQ), :] = jnp.dot(
                ob, wo_v[...].astype(BF16),
                preferred_element_type=F32).astype(BF16)
            for t, rdma in rs_rdmas:
                lo, hi = b * SQ // CH, (b + 1) * SQ // CH
                pl.when(jnp.logical_and(t >= lo, t < hi))(rdma.start)

        red = part_ref[pl.ds(CH * me, CH), :].astype(F32)
        for k in range(1, N_DEV):
            recv = pltpu.make_async_remote_copy(
                src_ref=rs_buf.at[k], dst_ref=rs_buf.at[k],
                send_sem=rs_send.at[0], recv_sem=rs_recv.at[k],
                device_id=(me,), device_id_type=pl.DeviceIdType.MESH,
            )
            recv.wait_recv()
            red = red + rs_buf[k].astype(F32)
        red_ref[...] = red.astype(BF16)

        for d in range(1, N_DEV):
            t = lax.rem(me + d, N_DEV)
            rdma = pltpu.make_async_remote_copy(
                src_ref=red_ref,
                dst_ref=ag_buf.at[N_DEV - d],
                send_sem=ag_send.at[d - 1],
                recv_sem=ag_recv.at[N_DEV - d],
                device_id=(t,),
                device_id_type=pl.DeviceIdType.MESH,
            )
            rdma.start()
            sends.append(rdma)

        out_ref[pl.ds(CH * me, CH), :] = red

        for k in range(1, N_DEV):
            recv = pltpu.make_async_remote_copy(
                src_ref=red_ref, dst_ref=ag_buf.at[k],
                send_sem=ag_send.at[0], recv_sem=ag_recv.at[k],
                device_id=(me,), device_id_type=pl.DeviceIdType.MESH,
            )
            recv.wait_recv()
            p = lax.rem(me + k, N_DEV)
            out_ref[pl.ds(CH * p, CH), :] = ag_buf[k].astype(F32)

        for rdma in sends:
            rdma.wait_send()

    out = pl.pallas_call(
        body,
        out_shape=jax.ShapeDtypeStruct((ROWS, D), F32),
        in_specs=[pl.BlockSpec(memory_space=pl.ANY)] * 5,
        out_specs=pl.BlockSpec(memory_space=pltpu.VMEM),
        scratch_shapes=[
            pltpu.VMEM((B, SQ, D), F32),
            pltpu.VMEM((D, ROWS), F32),
            pltpu.VMEM((ROWS, D), F32),
            pltpu.VMEM((2, B, SKV, 2, DH), F32),
            pltpu.VMEM((ROWS, D), BF16),
            pltpu.VMEM((CH, D), BF16),
            pltpu.VMEM((N_DEV, CH, D), BF16),
            pltpu.VMEM((N_DEV, CH, D), BF16),
            pltpu.SemaphoreType.DMA((5,)),
            pltpu.SemaphoreType.DMA((N_DEV - 1,)),
            pltpu.SemaphoreType.DMA((N_DEV,)),
            pltpu.SemaphoreType.DMA((N_DEV - 1,)),
            pltpu.SemaphoreType.DMA((N_DEV,)),
        ],
        compiler_params=pltpu.CompilerParams(collective_id=0),
    )(x, Wq, Wo, K_ext, V_ext)
    return out.reshape(B, SQ, D)
